# baseline (speedup 1.0000x reference)
import functools

import jax
import jax.numpy as jnp
from jax import lax
from jax.experimental import pallas as pl
from jax.experimental.pallas import tpu as pltpu
from jax.experimental.pallas import tpu_sc as plsc

E = 8
TOP_K = 2
D_MODEL = 2048
D_FF = 4096
T = 8192

BT = 512
NBLK = T // BT
NSLOT = T * TOP_K
NTILE = NSLOT // BT + E - 1
NROW = NTILE * BT
FFB = 512
NFF = D_FF // FFB
NEG = -1e30


def _route_body(gl_ref, rout_ref, tot_ref, carry):
    b = pl.program_id(0)

    @pl.when(b == 0)
    def _():
        carry[...] = jnp.zeros_like(carry)

    logits = gl_ref[...]
    li = lax.broadcasted_iota(jnp.int32, (BT, 128), 1)

    m1 = jnp.max(logits, axis=1, keepdims=True)
    a1 = jnp.min(jnp.where(logits == m1, li, 128), axis=1, keepdims=True)
    sel1 = li == a1
    logits2 = jnp.where(sel1, NEG, logits)
    m2 = jnp.max(logits2, axis=1, keepdims=True)
    a2 = jnp.min(jnp.where(logits2 == m2, li, 128), axis=1, keepdims=True)
    sel2 = li == a2

    e = jnp.exp(m2 - m1)
    w1v = 1.0 / (1.0 + e)
    w2v = 1.0 - w1v

    oh1 = sel1.astype(jnp.float32)
    oh2 = sel2.astype(jnp.float32)
    ri = lax.broadcasted_iota(jnp.int32, (BT, BT), 0)
    ci = lax.broadcasted_iota(jnp.int32, (BT, BT), 1)
    tril = (ci < ri).astype(jnp.float32)
    ex1 = jnp.dot(tril, oh1, preferred_element_type=jnp.float32)
    ex2 = jnp.dot(tril, oh2, preferred_element_type=jnp.float32)
    cnt1 = jnp.sum(oh1, axis=0, keepdims=True)
    cnt2 = jnp.sum(oh2, axis=0, keepdims=True)
    c0 = carry[...]
    rank1 = jnp.sum(oh1 * (c0 + ex1), axis=1, keepdims=True)
    rank2 = jnp.sum(oh2 * (c0 + cnt1 + ex2), axis=1, keepdims=True)
    cnew = c0 + cnt1 + cnt2
    carry[...] = cnew
    tot_ref[...] = cnew.reshape(1, 1, 128)

    a1f = a1.astype(jnp.float32)
    a2f = a2.astype(jnp.float32)
    packed = (jnp.where(li == 0, a1f, 0.0) + jnp.where(li == 1, a2f, 0.0)
              + jnp.where(li == 2, rank1, 0.0) + jnp.where(li == 3, rank2, 0.0)
              + jnp.where(li == 4, w1v, 0.0) + jnp.where(li == 5, w2v, 0.0))
    rout_ref[...] = packed


def _route(glp):
    return pl.pallas_call(
        _route_body,
        grid=(NBLK,),
        in_specs=[
            pl.BlockSpec((BT, 128), lambda b: (b, 0)),
        ],
        out_specs=[
            pl.BlockSpec((BT, 128), lambda b: (b, 0)),
            pl.BlockSpec((1, 1, 128), lambda b: (b, 0, 0)),
        ],
        out_shape=[
            jax.ShapeDtypeStruct((T, 128), jnp.float32),
            jax.ShapeDtypeStruct((NBLK, 1, 128), jnp.float32),
        ],
        scratch_shapes=[pltpu.VMEM((1, 128), jnp.float32)],
    )(glp)


def _sc_mesh():
    return plsc.VectorSubcoreMesh(core_axis_name="c", subcore_axis_name="s")


def _dispatch_body(x_hbm, e1_hbm, e2_hbm, r1_hbm, r2_hbm, base_hbm, xg_hbm,
                   e1l, e2l, r1l, r2l, basel, xbuf, sem):
    wid = lax.axis_index("s") * 2 + lax.axis_index("c")
    tpw = T // 32
    tok0 = wid * tpw
    pltpu.sync_copy(base_hbm, basel)
    pltpu.sync_copy(e1_hbm.at[pl.ds(tok0, tpw)], e1l)
    pltpu.sync_copy(e2_hbm.at[pl.ds(tok0, tpw)], e2l)
    pltpu.sync_copy(r1_hbm.at[pl.ds(tok0, tpw)], r1l)
    pltpu.sync_copy(r2_hbm.at[pl.ds(tok0, tpw)], r2l)
    for j in range(tpw // 16):
        t0 = j * 16
        ev1 = e1l[pl.ds(t0, 16)]
        ev2 = e2l[pl.ds(t0, 16)]
        rv1 = r1l[pl.ds(t0, 16)]
        rv2 = r2l[pl.ds(t0, 16)]
        pos1 = plsc.load_gather(basel, [ev1]) + rv1
        pos2 = plsc.load_gather(basel, [ev2]) + rv2
        pltpu.sync_copy(x_hbm.at[pl.ds(tok0 + t0, 16)], xbuf)
        pltpu.async_copy(xbuf, xg_hbm.at[pos1], sem).wait()
        pltpu.async_copy(xbuf, xg_hbm.at[pos2], sem).wait()


def _dispatch(x, e1, e2, r1, r2, base16):
    tpw = T // 32
    fn = functools.partial(
        pl.kernel, mesh=_sc_mesh(),
        out_type=jax.ShapeDtypeStruct((NROW, D_MODEL), jnp.float32),
        scratch_types=[
            pltpu.VMEM((tpw,), jnp.int32),
            pltpu.VMEM((tpw,), jnp.int32),
            pltpu.VMEM((tpw,), jnp.int32),
            pltpu.VMEM((tpw,), jnp.int32),
            pltpu.VMEM((16,), jnp.int32),
            pltpu.VMEM((16, D_MODEL), jnp.float32),
            pltpu.SemaphoreType.DMA,
        ],
        compiler_params=pltpu.CompilerParams(needs_layout_passes=False),
    )(_dispatch_body)
    return fn(x, e1, e2, r1, r2, base16)


def _gmm_body(grp_ref, valid_ref, xg_ref, w1_ref, b1_ref, w2_ref, b2_ref,
              og_ref):
    p = pl.program_id(0)
    ff = pl.program_id(1)

    @pl.when(ff == 0)
    def _():
        og_ref[...] = jnp.broadcast_to(b2_ref[0], (BT, D_MODEL))

    @pl.when(valid_ref[p] == 1)
    def _():
        x = xg_ref[...].astype(jnp.bfloat16)
        h = jnp.dot(x, w1_ref[0], preferred_element_type=jnp.float32)
        h = h + b1_ref[0]
        h = h * jax.nn.sigmoid(h)
        og_ref[...] += jnp.dot(h.astype(jnp.bfloat16), w2_ref[0],
                               preferred_element_type=jnp.float32)


def _gmm(grp, valid, xg, W1, b1r, W2, b2r):
    grid_spec = pltpu.PrefetchScalarGridSpec(
        num_scalar_prefetch=2,
        grid=(NTILE, NFF),
        in_specs=[
            pl.BlockSpec((BT, D_MODEL), lambda p, ff, g, v: (p, 0)),
            pl.BlockSpec((1, D_MODEL, FFB), lambda p, ff, g, v: (g[p], 0, ff)),
            pl.BlockSpec((1, 1, FFB), lambda p, ff, g, v: (g[p], 0, ff)),
            pl.BlockSpec((1, FFB, D_MODEL), lambda p, ff, g, v: (g[p], ff, 0)),
            pl.BlockSpec((1, 1, D_MODEL), lambda p, ff, g, v: (g[p], 0, 0)),
        ],
        out_specs=pl.BlockSpec((BT, D_MODEL), lambda p, ff, g, v: (p, 0)),
    )
    return pl.pallas_call(
        _gmm_body,
        grid_spec=grid_spec,
        out_shape=jax.ShapeDtypeStruct((NROW, D_MODEL), jnp.float32),
    )(grp, valid, xg, W1, b1r, W2, b2r)


def _combine_body(og_hbm, e1_hbm, e2_hbm, r1_hbm, r2_hbm, w1_hbm, w2_hbm,
                  base_hbm, out_hbm,
                  e1l, e2l, r1l, r2l, w1l, w2l, basel, posb1, posb2,
                  buf1, buf2, obuf, sem):
    wid = lax.axis_index("s") * 2 + lax.axis_index("c")
    tpw = T // 32
    tok0 = wid * tpw
    pltpu.sync_copy(base_hbm, basel)
    pltpu.sync_copy(e1_hbm.at[pl.ds(tok0, tpw)], e1l)
    pltpu.sync_copy(e2_hbm.at[pl.ds(tok0, tpw)], e2l)
    pltpu.sync_copy(r1_hbm.at[pl.ds(tok0, tpw)], r1l)
    pltpu.sync_copy(r2_hbm.at[pl.ds(tok0, tpw)], r2l)
    pltpu.sync_copy(w1_hbm.at[pl.ds(tok0, tpw)], w1l)
    pltpu.sync_copy(w2_hbm.at[pl.ds(tok0, tpw)], w2l)

    def jbody(j, _):
        t0 = j * 16
        ev1 = e1l[pl.ds(t0, 16)]
        ev2 = e2l[pl.ds(t0, 16)]
        rv1 = r1l[pl.ds(t0, 16)]
        rv2 = r2l[pl.ds(t0, 16)]
        posb1[...] = plsc.load_gather(basel, [ev1]) + rv1
        posb2[...] = plsc.load_gather(basel, [ev2]) + rv2
        cp1 = pltpu.async_copy(og_hbm.at[posb1], buf1, sem)
        cp2 = pltpu.async_copy(og_hbm.at[posb2], buf2, sem)
        cp1.wait()
        cp2.wait()
        for r in range(16):
            idxv = jnp.zeros((16,), jnp.int32) + (t0 + r)
            ws1 = plsc.load_gather(w1l, [idxv])
            ws2 = plsc.load_gather(w2l, [idxv])

            def cbody(c, _):
                for u in range(8):
                    sl = pl.ds(c * 128 + u * 16, 16)
                    obuf[r, sl] = buf1[r, sl] * ws1 + buf2[r, sl] * ws2
                return 0

            lax.fori_loop(0, D_MODEL // 128, cbody, 0)
        pltpu.sync_copy(obuf, out_hbm.at[pl.ds(tok0 + t0, 16)])
        return 0

    lax.fori_loop(0, tpw // 16, jbody, 0)


def _combine(og, e1, e2, r1, r2, w1, w2, base16):
    tpw = T // 32
    fn = functools.partial(
        pl.kernel, mesh=_sc_mesh(),
        out_type=jax.ShapeDtypeStruct((T, D_MODEL), jnp.float32),
        scratch_types=[
            pltpu.VMEM((tpw,), jnp.int32),
            pltpu.VMEM((tpw,), jnp.int32),
            pltpu.VMEM((tpw,), jnp.int32),
            pltpu.VMEM((tpw,), jnp.int32),
            pltpu.VMEM((tpw,), jnp.float32),
            pltpu.VMEM((tpw,), jnp.float32),
            pltpu.VMEM((16,), jnp.int32),
            pltpu.VMEM((16,), jnp.int32),
            pltpu.VMEM((16,), jnp.int32),
            pltpu.VMEM((16, D_MODEL), jnp.float32),
            pltpu.VMEM((16, D_MODEL), jnp.float32),
            pltpu.VMEM((16, D_MODEL), jnp.float32),
            pltpu.SemaphoreType.DMA,
        ],
        compiler_params=pltpu.CompilerParams(needs_layout_passes=False),
    )(_combine_body)
    return fn(og, e1, e2, r1, r2, w1, w2, base16)


def kernel(inputs, Wg, bg, W1, b1, W2, b2):
    i32 = jnp.int32
    gl = inputs @ Wg + bg
    glp = jnp.concatenate(
        [gl, jnp.full((T, 128 - E), NEG, jnp.float32)], axis=1)

    rout, totf = _route(glp)

    e1 = rout[:, 0].astype(i32)
    e2 = rout[:, 1].astype(i32)
    r1 = rout[:, 2].astype(i32)
    r2 = rout[:, 3].astype(i32)
    w1 = rout[:, 4]
    w2 = rout[:, 5]

    sizes = totf[NBLK - 1, 0, :E].astype(i32)
    tpe = (sizes + BT - 1) // BT
    tstart = jnp.concatenate([jnp.zeros((1,), i32), jnp.cumsum(tpe)])[:E]
    base16 = jnp.pad(tstart * BT, (0, 16 - E)).astype(i32)
    grp = (jnp.arange(NTILE, dtype=i32)[:, None]
           >= tstart[None, :]).sum(axis=1).astype(i32) - 1
    valid = (jnp.arange(NTILE, dtype=i32) < jnp.sum(tpe)).astype(i32)

    xg = _dispatch(inputs, e1, e2, r1, r2, base16)
    b1r = b1.reshape(E, 1, D_FF)
    b2r = b2.reshape(E, 1, D_MODEL)
    og = _gmm(grp, valid, xg, W1.astype(jnp.bfloat16), b1r,
              W2.astype(jnp.bfloat16), b2r)
    return _combine(og, e1, e2, r1, r2, w1, w2, base16)


# speedup vs baseline: 1.0365x; 1.0365x over previous
import functools

import jax
import jax.numpy as jnp
from jax import lax
from jax.experimental import pallas as pl
from jax.experimental.pallas import tpu as pltpu
from jax.experimental.pallas import tpu_sc as plsc

E = 8
TOP_K = 2
D_MODEL = 2048
D_FF = 4096
T = 8192

BT = 512
NBLK = T // BT
NSLOT = T * TOP_K
NTILE = NSLOT // BT + E - 1
NROW = NTILE * BT
FFB = 512
NFF = D_FF // FFB
NEG = -1e30


def _route_body(gl_ref, rout_ref, tot_ref, carry):
    b = pl.program_id(0)

    @pl.when(b == 0)
    def _():
        carry[...] = jnp.zeros_like(carry)

    logits = gl_ref[...]
    li = lax.broadcasted_iota(jnp.int32, (BT, 128), 1)

    m1 = jnp.max(logits, axis=1, keepdims=True)
    a1 = jnp.min(jnp.where(logits == m1, li, 128), axis=1, keepdims=True)
    sel1 = li == a1
    logits2 = jnp.where(sel1, NEG, logits)
    m2 = jnp.max(logits2, axis=1, keepdims=True)
    a2 = jnp.min(jnp.where(logits2 == m2, li, 128), axis=1, keepdims=True)
    sel2 = li == a2

    e = jnp.exp(m2 - m1)
    w1v = 1.0 / (1.0 + e)
    w2v = 1.0 - w1v

    oh1 = sel1.astype(jnp.float32)
    oh2 = sel2.astype(jnp.float32)
    ri = lax.broadcasted_iota(jnp.int32, (BT, BT), 0)
    ci = lax.broadcasted_iota(jnp.int32, (BT, BT), 1)
    tril = (ci < ri).astype(jnp.float32)
    ex1 = jnp.dot(tril, oh1, preferred_element_type=jnp.float32)
    ex2 = jnp.dot(tril, oh2, preferred_element_type=jnp.float32)
    cnt1 = jnp.sum(oh1, axis=0, keepdims=True)
    cnt2 = jnp.sum(oh2, axis=0, keepdims=True)
    c0 = carry[...]
    rank1 = jnp.sum(oh1 * (c0 + ex1), axis=1, keepdims=True)
    rank2 = jnp.sum(oh2 * (c0 + cnt1 + ex2), axis=1, keepdims=True)
    cnew = c0 + cnt1 + cnt2
    carry[...] = cnew
    tot_ref[...] = cnew.reshape(1, 1, 128)

    a1f = a1.astype(jnp.float32)
    a2f = a2.astype(jnp.float32)
    packed = (jnp.where(li == 0, a1f, 0.0) + jnp.where(li == 1, a2f, 0.0)
              + jnp.where(li == 2, rank1, 0.0) + jnp.where(li == 3, rank2, 0.0)
              + jnp.where(li == 4, w1v, 0.0) + jnp.where(li == 5, w2v, 0.0))
    rout_ref[...] = packed


def _route(glp):
    return pl.pallas_call(
        _route_body,
        grid=(NBLK,),
        in_specs=[
            pl.BlockSpec((BT, 128), lambda b: (b, 0)),
        ],
        out_specs=[
            pl.BlockSpec((BT, 128), lambda b: (b, 0)),
            pl.BlockSpec((1, 1, 128), lambda b: (b, 0, 0)),
        ],
        out_shape=[
            jax.ShapeDtypeStruct((T, 128), jnp.float32),
            jax.ShapeDtypeStruct((NBLK, 1, 128), jnp.float32),
        ],
        scratch_shapes=[pltpu.VMEM((1, 128), jnp.float32)],
    )(glp)


def _sc_mesh():
    return plsc.VectorSubcoreMesh(core_axis_name="c", subcore_axis_name="s")


def _dispatch_body(x_hbm, e1_hbm, e2_hbm, r1_hbm, r2_hbm, base_hbm, xg_hbm,
                   e1l, e2l, r1l, r2l, basel, xbuf, sem):
    wid = lax.axis_index("s") * 2 + lax.axis_index("c")
    tpw = T // 32
    tok0 = wid * tpw
    pltpu.sync_copy(base_hbm, basel)
    pltpu.sync_copy(e1_hbm.at[pl.ds(tok0, tpw)], e1l)
    pltpu.sync_copy(e2_hbm.at[pl.ds(tok0, tpw)], e2l)
    pltpu.sync_copy(r1_hbm.at[pl.ds(tok0, tpw)], r1l)
    pltpu.sync_copy(r2_hbm.at[pl.ds(tok0, tpw)], r2l)
    for j in range(tpw // 16):
        t0 = j * 16
        ev1 = e1l[pl.ds(t0, 16)]
        ev2 = e2l[pl.ds(t0, 16)]
        rv1 = r1l[pl.ds(t0, 16)]
        rv2 = r2l[pl.ds(t0, 16)]
        pos1 = plsc.load_gather(basel, [ev1]) + rv1
        pos2 = plsc.load_gather(basel, [ev2]) + rv2
        pltpu.sync_copy(x_hbm.at[pl.ds(tok0 + t0, 16)], xbuf)
        pltpu.async_copy(xbuf, xg_hbm.at[pos1], sem).wait()
        pltpu.async_copy(xbuf, xg_hbm.at[pos2], sem).wait()


def _dispatch(x, e1, e2, r1, r2, base16):
    tpw = T // 32
    fn = functools.partial(
        pl.kernel, mesh=_sc_mesh(),
        out_type=jax.ShapeDtypeStruct((NROW, D_MODEL), jnp.float32),
        scratch_types=[
            pltpu.VMEM((tpw,), jnp.int32),
            pltpu.VMEM((tpw,), jnp.int32),
            pltpu.VMEM((tpw,), jnp.int32),
            pltpu.VMEM((tpw,), jnp.int32),
            pltpu.VMEM((16,), jnp.int32),
            pltpu.VMEM((16, D_MODEL), jnp.float32),
            pltpu.SemaphoreType.DMA,
        ],
        compiler_params=pltpu.CompilerParams(needs_layout_passes=False),
    )(_dispatch_body)
    return fn(x, e1, e2, r1, r2, base16)


def _gmm_body(grp_ref, valid_ref, xg_ref, w1_ref, b1_ref, w2_ref, b2_ref,
              og_ref):
    p = pl.program_id(0)
    ff = pl.program_id(1)

    @pl.when(ff == 0)
    def _():
        og_ref[...] = jnp.broadcast_to(b2_ref[0], (BT, D_MODEL))

    @pl.when(valid_ref[p] == 1)
    def _():
        x = xg_ref[...].astype(jnp.bfloat16)
        h = jnp.dot(x, w1_ref[0].astype(jnp.bfloat16),
                    preferred_element_type=jnp.float32)
        h = h + b1_ref[0]
        h = h * jax.nn.sigmoid(h)
        og_ref[...] += jnp.dot(h.astype(jnp.bfloat16),
                               w2_ref[0].astype(jnp.bfloat16),
                               preferred_element_type=jnp.float32)


def _gmm(grp, valid, xg, W1, b1r, W2, b2r):
    grid_spec = pltpu.PrefetchScalarGridSpec(
        num_scalar_prefetch=2,
        grid=(NTILE, NFF),
        in_specs=[
            pl.BlockSpec((BT, D_MODEL), lambda p, ff, g, v: (p, 0)),
            pl.BlockSpec((1, D_MODEL, FFB), lambda p, ff, g, v: (g[p], 0, ff)),
            pl.BlockSpec((1, 1, FFB), lambda p, ff, g, v: (g[p], 0, ff)),
            pl.BlockSpec((1, FFB, D_MODEL), lambda p, ff, g, v: (g[p], ff, 0)),
            pl.BlockSpec((1, 1, D_MODEL), lambda p, ff, g, v: (g[p], 0, 0)),
        ],
        out_specs=pl.BlockSpec((BT, D_MODEL), lambda p, ff, g, v: (p, 0)),
    )
    return pl.pallas_call(
        _gmm_body,
        grid_spec=grid_spec,
        out_shape=jax.ShapeDtypeStruct((NROW, D_MODEL), jnp.float32),
    )(grp, valid, xg, W1, b1r, W2, b2r)


def _combine_body(og_hbm, e1_hbm, e2_hbm, r1_hbm, r2_hbm, w1_hbm, w2_hbm,
                  base_hbm, out_hbm,
                  e1l, e2l, r1l, r2l, w1l, w2l, basel, posb1, posb2,
                  buf1, buf2, obuf, sem):
    wid = lax.axis_index("s") * 2 + lax.axis_index("c")
    tpw = T // 32
    tok0 = wid * tpw
    pltpu.sync_copy(base_hbm, basel)
    pltpu.sync_copy(e1_hbm.at[pl.ds(tok0, tpw)], e1l)
    pltpu.sync_copy(e2_hbm.at[pl.ds(tok0, tpw)], e2l)
    pltpu.sync_copy(r1_hbm.at[pl.ds(tok0, tpw)], r1l)
    pltpu.sync_copy(r2_hbm.at[pl.ds(tok0, tpw)], r2l)
    pltpu.sync_copy(w1_hbm.at[pl.ds(tok0, tpw)], w1l)
    pltpu.sync_copy(w2_hbm.at[pl.ds(tok0, tpw)], w2l)

    def jbody(j, _):
        t0 = j * 16
        ev1 = e1l[pl.ds(t0, 16)]
        ev2 = e2l[pl.ds(t0, 16)]
        rv1 = r1l[pl.ds(t0, 16)]
        rv2 = r2l[pl.ds(t0, 16)]
        posb1[...] = plsc.load_gather(basel, [ev1]) + rv1
        posb2[...] = plsc.load_gather(basel, [ev2]) + rv2
        cp1 = pltpu.async_copy(og_hbm.at[posb1], buf1, sem)
        cp2 = pltpu.async_copy(og_hbm.at[posb2], buf2, sem)
        cp1.wait()
        cp2.wait()
        for r in range(16):
            idxv = jnp.zeros((16,), jnp.int32) + (t0 + r)
            ws1 = plsc.load_gather(w1l, [idxv])
            ws2 = plsc.load_gather(w2l, [idxv])

            def cbody(c, _):
                for u in range(8):
                    sl = pl.ds(c * 128 + u * 16, 16)
                    obuf[r, sl] = buf1[r, sl] * ws1 + buf2[r, sl] * ws2
                return 0

            lax.fori_loop(0, D_MODEL // 128, cbody, 0)
        pltpu.sync_copy(obuf, out_hbm.at[pl.ds(tok0 + t0, 16)])
        return 0

    lax.fori_loop(0, tpw // 16, jbody, 0)


def _combine(og, e1, e2, r1, r2, w1, w2, base16):
    tpw = T // 32
    fn = functools.partial(
        pl.kernel, mesh=_sc_mesh(),
        out_type=jax.ShapeDtypeStruct((T, D_MODEL), jnp.float32),
        scratch_types=[
            pltpu.VMEM((tpw,), jnp.int32),
            pltpu.VMEM((tpw,), jnp.int32),
            pltpu.VMEM((tpw,), jnp.int32),
            pltpu.VMEM((tpw,), jnp.int32),
            pltpu.VMEM((tpw,), jnp.float32),
            pltpu.VMEM((tpw,), jnp.float32),
            pltpu.VMEM((16,), jnp.int32),
            pltpu.VMEM((16,), jnp.int32),
            pltpu.VMEM((16,), jnp.int32),
            pltpu.VMEM((16, D_MODEL), jnp.float32),
            pltpu.VMEM((16, D_MODEL), jnp.float32),
            pltpu.VMEM((16, D_MODEL), jnp.float32),
            pltpu.SemaphoreType.DMA,
        ],
        compiler_params=pltpu.CompilerParams(needs_layout_passes=False),
    )(_combine_body)
    return fn(og, e1, e2, r1, r2, w1, w2, base16)


def kernel(inputs, Wg, bg, W1, b1, W2, b2):
    i32 = jnp.int32
    gl = inputs @ Wg + bg
    glp = jnp.concatenate(
        [gl, jnp.full((T, 128 - E), NEG, jnp.float32)], axis=1)

    rout, totf = _route(glp)

    e1 = rout[:, 0].astype(i32)
    e2 = rout[:, 1].astype(i32)
    r1 = rout[:, 2].astype(i32)
    r2 = rout[:, 3].astype(i32)
    w1 = rout[:, 4]
    w2 = rout[:, 5]

    sizes = totf[NBLK - 1, 0, :E].astype(i32)
    tpe = (sizes + BT - 1) // BT
    tstart = jnp.concatenate([jnp.zeros((1,), i32), jnp.cumsum(tpe)])[:E]
    base16 = jnp.pad(tstart * BT, (0, 16 - E)).astype(i32)
    grp = (jnp.arange(NTILE, dtype=i32)[:, None]
           >= tstart[None, :]).sum(axis=1).astype(i32) - 1
    valid = (jnp.arange(NTILE, dtype=i32) < jnp.sum(tpe)).astype(i32)

    xg = _dispatch(inputs, e1, e2, r1, r2, base16)
    b1r = b1.reshape(E, 1, D_FF)
    b2r = b2.reshape(E, 1, D_MODEL)
    og = _gmm(grp, valid, xg, W1, b1r, W2, b2r)
    return _combine(og, e1, e2, r1, r2, w1, w2, base16)


# speedup vs baseline: 1.1201x; 1.0807x over previous
import functools

import jax
import jax.numpy as jnp
from jax import lax
from jax.experimental import pallas as pl
from jax.experimental.pallas import tpu as pltpu
from jax.experimental.pallas import tpu_sc as plsc

E = 8
TOP_K = 2
D_MODEL = 2048
D_FF = 4096
T = 8192

BT = 512
NBLK = T // BT
NSLOT = T * TOP_K
NTILE = NSLOT // BT + E - 1
NROW = NTILE * BT
FFB = 1024
NFF = D_FF // FFB
NEG = -1e30


def _route_body(gl_ref, rout_ref, tot_ref, carry):
    b = pl.program_id(0)

    @pl.when(b == 0)
    def _():
        carry[...] = jnp.zeros_like(carry)

    logits = gl_ref[...]
    li = lax.broadcasted_iota(jnp.int32, (BT, 128), 1)

    m1 = jnp.max(logits, axis=1, keepdims=True)
    a1 = jnp.min(jnp.where(logits == m1, li, 128), axis=1, keepdims=True)
    sel1 = li == a1
    logits2 = jnp.where(sel1, NEG, logits)
    m2 = jnp.max(logits2, axis=1, keepdims=True)
    a2 = jnp.min(jnp.where(logits2 == m2, li, 128), axis=1, keepdims=True)
    sel2 = li == a2

    e = jnp.exp(m2 - m1)
    w1v = 1.0 / (1.0 + e)
    w2v = 1.0 - w1v

    oh1 = sel1.astype(jnp.float32)
    oh2 = sel2.astype(jnp.float32)
    ri = lax.broadcasted_iota(jnp.int32, (BT, BT), 0)
    ci = lax.broadcasted_iota(jnp.int32, (BT, BT), 1)
    tril = (ci < ri).astype(jnp.float32)
    ex1 = jnp.dot(tril, oh1, preferred_element_type=jnp.float32)
    ex2 = jnp.dot(tril, oh2, preferred_element_type=jnp.float32)
    cnt1 = jnp.sum(oh1, axis=0, keepdims=True)
    cnt2 = jnp.sum(oh2, axis=0, keepdims=True)
    c0 = carry[...]
    rank1 = jnp.sum(oh1 * (c0 + ex1), axis=1, keepdims=True)
    rank2 = jnp.sum(oh2 * (c0 + cnt1 + ex2), axis=1, keepdims=True)
    cnew = c0 + cnt1 + cnt2
    carry[...] = cnew
    tot_ref[...] = cnew.reshape(1, 1, 128)

    a1f = a1.astype(jnp.float32)
    a2f = a2.astype(jnp.float32)
    packed = (jnp.where(li == 0, a1f, 0.0) + jnp.where(li == 1, a2f, 0.0)
              + jnp.where(li == 2, rank1, 0.0) + jnp.where(li == 3, rank2, 0.0)
              + jnp.where(li == 4, w1v, 0.0) + jnp.where(li == 5, w2v, 0.0))
    rout_ref[...] = packed


def _route(glp):
    return pl.pallas_call(
        _route_body,
        grid=(NBLK,),
        in_specs=[
            pl.BlockSpec((BT, 128), lambda b: (b, 0)),
        ],
        out_specs=[
            pl.BlockSpec((BT, 128), lambda b: (b, 0)),
            pl.BlockSpec((1, 1, 128), lambda b: (b, 0, 0)),
        ],
        out_shape=[
            jax.ShapeDtypeStruct((T, 128), jnp.float32),
            jax.ShapeDtypeStruct((NBLK, 1, 128), jnp.float32),
        ],
        scratch_shapes=[pltpu.VMEM((1, 128), jnp.float32)],
    )(glp)


def _sc_mesh():
    return plsc.VectorSubcoreMesh(core_axis_name="c", subcore_axis_name="s")


def _dispatch_body(x_hbm, e1_hbm, e2_hbm, r1_hbm, r2_hbm, base_hbm, xg_hbm,
                   e1l, e2l, r1l, r2l, basel, xbuf, sem):
    wid = lax.axis_index("s") * 2 + lax.axis_index("c")
    tpw = T // 32
    tok0 = wid * tpw
    pltpu.sync_copy(base_hbm, basel)
    pltpu.sync_copy(e1_hbm.at[pl.ds(tok0, tpw)], e1l)
    pltpu.sync_copy(e2_hbm.at[pl.ds(tok0, tpw)], e2l)
    pltpu.sync_copy(r1_hbm.at[pl.ds(tok0, tpw)], r1l)
    pltpu.sync_copy(r2_hbm.at[pl.ds(tok0, tpw)], r2l)
    for j in range(tpw // 16):
        t0 = j * 16
        ev1 = e1l[pl.ds(t0, 16)]
        ev2 = e2l[pl.ds(t0, 16)]
        rv1 = r1l[pl.ds(t0, 16)]
        rv2 = r2l[pl.ds(t0, 16)]
        pos1 = plsc.load_gather(basel, [ev1]) + rv1
        pos2 = plsc.load_gather(basel, [ev2]) + rv2
        pltpu.sync_copy(x_hbm.at[pl.ds(tok0 + t0, 16)], xbuf)
        pltpu.async_copy(xbuf, xg_hbm.at[pos1], sem).wait()
        pltpu.async_copy(xbuf, xg_hbm.at[pos2], sem).wait()


def _dispatch(x, e1, e2, r1, r2, base16):
    tpw = T // 32
    fn = functools.partial(
        pl.kernel, mesh=_sc_mesh(),
        out_type=jax.ShapeDtypeStruct((NROW, D_MODEL), jnp.float32),
        scratch_types=[
            pltpu.VMEM((tpw,), jnp.int32),
            pltpu.VMEM((tpw,), jnp.int32),
            pltpu.VMEM((tpw,), jnp.int32),
            pltpu.VMEM((tpw,), jnp.int32),
            pltpu.VMEM((16,), jnp.int32),
            pltpu.VMEM((16, D_MODEL), jnp.float32),
            pltpu.SemaphoreType.DMA,
        ],
        compiler_params=pltpu.CompilerParams(needs_layout_passes=False),
    )(_dispatch_body)
    return fn(x, e1, e2, r1, r2, base16)


def _gmm_body(grp_ref, valid_ref, xg_ref, w1_ref, b1_ref, w2_ref, b2_ref,
              og_ref):
    p = pl.program_id(0)
    ff = pl.program_id(1)

    @pl.when(ff == 0)
    def _():
        og_ref[...] = jnp.broadcast_to(b2_ref[0], (BT, D_MODEL))

    @pl.when(valid_ref[p] == 1)
    def _():
        x = xg_ref[...].astype(jnp.bfloat16)
        h = jnp.dot(x, w1_ref[0].astype(jnp.bfloat16),
                    preferred_element_type=jnp.float32)
        h = h + b1_ref[0]
        h = h * jax.nn.sigmoid(h)
        og_ref[...] += jnp.dot(h.astype(jnp.bfloat16),
                               w2_ref[0].astype(jnp.bfloat16),
                               preferred_element_type=jnp.float32)


def _gmm(grp, valid, xg, W1, b1r, W2, b2r):
    grid_spec = pltpu.PrefetchScalarGridSpec(
        num_scalar_prefetch=2,
        grid=(NTILE, NFF),
        in_specs=[
            pl.BlockSpec((BT, D_MODEL), lambda p, ff, g, v: (p, 0)),
            pl.BlockSpec((1, D_MODEL, FFB), lambda p, ff, g, v: (g[p], 0, ff)),
            pl.BlockSpec((1, 1, FFB), lambda p, ff, g, v: (g[p], 0, ff)),
            pl.BlockSpec((1, FFB, D_MODEL), lambda p, ff, g, v: (g[p], ff, 0)),
            pl.BlockSpec((1, 1, D_MODEL), lambda p, ff, g, v: (g[p], 0, 0)),
        ],
        out_specs=pl.BlockSpec((BT, D_MODEL), lambda p, ff, g, v: (p, 0)),
    )
    return pl.pallas_call(
        _gmm_body,
        grid_spec=grid_spec,
        out_shape=jax.ShapeDtypeStruct((NROW, D_MODEL), jnp.float32),
    )(grp, valid, xg, W1, b1r, W2, b2r)


def _combine_body(og_hbm, e1_hbm, e2_hbm, r1_hbm, r2_hbm, w1_hbm, w2_hbm,
                  base_hbm, out_hbm,
                  e1l, e2l, r1l, r2l, w1l, w2l, basel, posb1, posb2,
                  buf1, buf2, obuf, sem):
    wid = lax.axis_index("s") * 2 + lax.axis_index("c")
    tpw = T // 32
    tok0 = wid * tpw
    pltpu.sync_copy(base_hbm, basel)
    pltpu.sync_copy(e1_hbm.at[pl.ds(tok0, tpw)], e1l)
    pltpu.sync_copy(e2_hbm.at[pl.ds(tok0, tpw)], e2l)
    pltpu.sync_copy(r1_hbm.at[pl.ds(tok0, tpw)], r1l)
    pltpu.sync_copy(r2_hbm.at[pl.ds(tok0, tpw)], r2l)
    pltpu.sync_copy(w1_hbm.at[pl.ds(tok0, tpw)], w1l)
    pltpu.sync_copy(w2_hbm.at[pl.ds(tok0, tpw)], w2l)

    def jbody(j, _):
        t0 = j * 16
        ev1 = e1l[pl.ds(t0, 16)]
        ev2 = e2l[pl.ds(t0, 16)]
        rv1 = r1l[pl.ds(t0, 16)]
        rv2 = r2l[pl.ds(t0, 16)]
        posb1[...] = plsc.load_gather(basel, [ev1]) + rv1
        posb2[...] = plsc.load_gather(basel, [ev2]) + rv2
        cp1 = pltpu.async_copy(og_hbm.at[posb1], buf1, sem)
        cp2 = pltpu.async_copy(og_hbm.at[posb2], buf2, sem)
        cp1.wait()
        cp2.wait()
        for r in range(16):
            idxv = jnp.zeros((16,), jnp.int32) + (t0 + r)
            ws1 = plsc.load_gather(w1l, [idxv])
            ws2 = plsc.load_gather(w2l, [idxv])

            def cbody(c, _):
                for u in range(8):
                    sl = pl.ds(c * 128 + u * 16, 16)
                    obuf[r, sl] = buf1[r, sl] * ws1 + buf2[r, sl] * ws2
                return 0

            lax.fori_loop(0, D_MODEL // 128, cbody, 0)
        pltpu.sync_copy(obuf, out_hbm.at[pl.ds(tok0 + t0, 16)])
        return 0

    lax.fori_loop(0, tpw // 16, jbody, 0)


def _combine(og, e1, e2, r1, r2, w1, w2, base16):
    tpw = T // 32
    fn = functools.partial(
        pl.kernel, mesh=_sc_mesh(),
        out_type=jax.ShapeDtypeStruct((T, D_MODEL), jnp.float32),
        scratch_types=[
            pltpu.VMEM((tpw,), jnp.int32),
            pltpu.VMEM((tpw,), jnp.int32),
            pltpu.VMEM((tpw,), jnp.int32),
            pltpu.VMEM((tpw,), jnp.int32),
            pltpu.VMEM((tpw,), jnp.float32),
            pltpu.VMEM((tpw,), jnp.float32),
            pltpu.VMEM((16,), jnp.int32),
            pltpu.VMEM((16,), jnp.int32),
            pltpu.VMEM((16,), jnp.int32),
            pltpu.VMEM((16, D_MODEL), jnp.float32),
            pltpu.VMEM((16, D_MODEL), jnp.float32),
            pltpu.VMEM((16, D_MODEL), jnp.float32),
            pltpu.SemaphoreType.DMA,
        ],
        compiler_params=pltpu.CompilerParams(needs_layout_passes=False),
    )(_combine_body)
    return fn(og, e1, e2, r1, r2, w1, w2, base16)


def kernel(inputs, Wg, bg, W1, b1, W2, b2):
    i32 = jnp.int32
    gl = inputs @ Wg + bg
    glp = jnp.concatenate(
        [gl, jnp.full((T, 128 - E), NEG, jnp.float32)], axis=1)

    rout, totf = _route(glp)

    e1 = rout[:, 0].astype(i32)
    e2 = rout[:, 1].astype(i32)
    r1 = rout[:, 2].astype(i32)
    r2 = rout[:, 3].astype(i32)
    w1 = rout[:, 4]
    w2 = rout[:, 5]

    sizes = totf[NBLK - 1, 0, :E].astype(i32)
    tpe = (sizes + BT - 1) // BT
    tstart = jnp.concatenate([jnp.zeros((1,), i32), jnp.cumsum(tpe)])[:E]
    base16 = jnp.pad(tstart * BT, (0, 16 - E)).astype(i32)
    grp = (jnp.arange(NTILE, dtype=i32)[:, None]
           >= tstart[None, :]).sum(axis=1).astype(i32) - 1
    valid = (jnp.arange(NTILE, dtype=i32) < jnp.sum(tpe)).astype(i32)

    xg = _dispatch(inputs, e1, e2, r1, r2, base16)
    b1r = b1.reshape(E, 1, D_FF)
    b2r = b2.reshape(E, 1, D_MODEL)
    og = _gmm(grp, valid, xg, W1, b1r, W2, b2r)
    return _combine(og, e1, e2, r1, r2, w1, w2, base16)


# speedup vs baseline: 1.1321x; 1.0107x over previous
import functools

import jax
import jax.numpy as jnp
from jax import lax
from jax.experimental import pallas as pl
from jax.experimental.pallas import tpu as pltpu
from jax.experimental.pallas import tpu_sc as plsc

E = 8
TOP_K = 2
D_MODEL = 2048
D_FF = 4096
T = 8192

BT = 512
NBLK = T // BT
NSLOT = T * TOP_K
NTILE = NSLOT // BT + E - 1
NROW = NTILE * BT
FFB = 1024
NFF = D_FF // FFB
NEG = -1e30


def _route_body(gl_ref, rout_ref, tot_ref, carry):
    b = pl.program_id(0)

    @pl.when(b == 0)
    def _():
        carry[...] = jnp.zeros_like(carry)

    logits = gl_ref[...]
    li = lax.broadcasted_iota(jnp.int32, (BT, 128), 1)

    m1 = jnp.max(logits, axis=1, keepdims=True)
    a1 = jnp.min(jnp.where(logits == m1, li, 128), axis=1, keepdims=True)
    sel1 = li == a1
    logits2 = jnp.where(sel1, NEG, logits)
    m2 = jnp.max(logits2, axis=1, keepdims=True)
    a2 = jnp.min(jnp.where(logits2 == m2, li, 128), axis=1, keepdims=True)
    sel2 = li == a2

    e = jnp.exp(m2 - m1)
    w1v = 1.0 / (1.0 + e)
    w2v = 1.0 - w1v

    oh1 = sel1.astype(jnp.float32)
    oh2 = sel2.astype(jnp.float32)
    ri = lax.broadcasted_iota(jnp.int32, (BT, BT), 0)
    ci = lax.broadcasted_iota(jnp.int32, (BT, BT), 1)
    tril = (ci < ri).astype(jnp.float32)
    ex1 = jnp.dot(tril, oh1, preferred_element_type=jnp.float32)
    ex2 = jnp.dot(tril, oh2, preferred_element_type=jnp.float32)
    cnt1 = jnp.sum(oh1, axis=0, keepdims=True)
    cnt2 = jnp.sum(oh2, axis=0, keepdims=True)
    c0 = carry[...]
    rank1 = jnp.sum(oh1 * (c0 + ex1), axis=1, keepdims=True)
    rank2 = jnp.sum(oh2 * (c0 + cnt1 + ex2), axis=1, keepdims=True)
    cnew = c0 + cnt1 + cnt2
    carry[...] = cnew
    tot_ref[...] = cnew.reshape(1, 1, 128)

    a1f = a1.astype(jnp.float32)
    a2f = a2.astype(jnp.float32)
    packed = (jnp.where(li == 0, a1f, 0.0) + jnp.where(li == 1, a2f, 0.0)
              + jnp.where(li == 2, rank1, 0.0) + jnp.where(li == 3, rank2, 0.0)
              + jnp.where(li == 4, w1v, 0.0) + jnp.where(li == 5, w2v, 0.0))
    rout_ref[...] = packed


def _route(glp):
    return pl.pallas_call(
        _route_body,
        grid=(NBLK,),
        in_specs=[
            pl.BlockSpec((BT, 128), lambda b: (b, 0)),
        ],
        out_specs=[
            pl.BlockSpec((BT, 128), lambda b: (b, 0)),
            pl.BlockSpec((1, 1, 128), lambda b: (b, 0, 0)),
        ],
        out_shape=[
            jax.ShapeDtypeStruct((T, 128), jnp.float32),
            jax.ShapeDtypeStruct((NBLK, 1, 128), jnp.float32),
        ],
        scratch_shapes=[pltpu.VMEM((1, 128), jnp.float32)],
    )(glp)


def _sc_mesh():
    return plsc.VectorSubcoreMesh(core_axis_name="c", subcore_axis_name="s")


def _rowcol(routl, rows, col):
    return plsc.load_gather(routl, [rows, jnp.full((16,), col, jnp.int32)])


def _dispatch_body(x_hbm, rout_hbm, base_hbm, xg_hbm, routl, basel, xbuf, sem):
    wid = lax.axis_index("s") * 2 + lax.axis_index("c")
    tpw = T // 32
    tok0 = wid * tpw
    pltpu.sync_copy(base_hbm, basel)
    pltpu.sync_copy(rout_hbm.at[pl.ds(tok0, tpw)], routl)
    for j in range(tpw // 16):
        t0 = j * 16
        rows = jnp.arange(16, dtype=jnp.int32) + t0
        ev1 = _rowcol(routl, rows, 0).astype(jnp.int32)
        ev2 = _rowcol(routl, rows, 1).astype(jnp.int32)
        rv1 = _rowcol(routl, rows, 2).astype(jnp.int32)
        rv2 = _rowcol(routl, rows, 3).astype(jnp.int32)
        pos1 = plsc.load_gather(basel, [ev1]) + rv1
        pos2 = plsc.load_gather(basel, [ev2]) + rv2
        pltpu.sync_copy(x_hbm.at[pl.ds(tok0 + t0, 16)], xbuf)
        pltpu.async_copy(xbuf, xg_hbm.at[pos1], sem).wait()
        pltpu.async_copy(xbuf, xg_hbm.at[pos2], sem).wait()


def _dispatch(x, rout, base16):
    tpw = T // 32
    fn = functools.partial(
        pl.kernel, mesh=_sc_mesh(),
        out_type=jax.ShapeDtypeStruct((NROW, D_MODEL), jnp.float32),
        scratch_types=[
            pltpu.VMEM((tpw, 128), jnp.float32),
            pltpu.VMEM((16,), jnp.int32),
            pltpu.VMEM((16, D_MODEL), jnp.float32),
            pltpu.SemaphoreType.DMA,
        ],
        compiler_params=pltpu.CompilerParams(needs_layout_passes=False),
    )(_dispatch_body)
    return fn(x, rout, base16)


def _gmm_body(grp_ref, valid_ref, xg_ref, w1_ref, b1_ref, w2_ref, b2_ref,
              og_ref):
    p = pl.program_id(0)
    ff = pl.program_id(1)

    @pl.when(ff == 0)
    def _():
        og_ref[...] = jnp.broadcast_to(b2_ref[0], (BT, D_MODEL))

    @pl.when(valid_ref[p] == 1)
    def _():
        x = xg_ref[...].astype(jnp.bfloat16)
        h = jnp.dot(x, w1_ref[0].astype(jnp.bfloat16),
                    preferred_element_type=jnp.float32)
        h = h + b1_ref[0]
        h = h * jax.nn.sigmoid(h)
        og_ref[...] += jnp.dot(h.astype(jnp.bfloat16),
                               w2_ref[0].astype(jnp.bfloat16),
                               preferred_element_type=jnp.float32)


def _gmm(grp, valid, xg, W1, b1r, W2, b2r):
    grid_spec = pltpu.PrefetchScalarGridSpec(
        num_scalar_prefetch=2,
        grid=(NTILE, NFF),
        in_specs=[
            pl.BlockSpec((BT, D_MODEL), lambda p, ff, g, v: (p, 0)),
            pl.BlockSpec((1, D_MODEL, FFB), lambda p, ff, g, v: (g[p], 0, ff)),
            pl.BlockSpec((1, 1, FFB), lambda p, ff, g, v: (g[p], 0, ff)),
            pl.BlockSpec((1, FFB, D_MODEL), lambda p, ff, g, v: (g[p], ff, 0)),
            pl.BlockSpec((1, 1, D_MODEL), lambda p, ff, g, v: (g[p], 0, 0)),
        ],
        out_specs=pl.BlockSpec((BT, D_MODEL), lambda p, ff, g, v: (p, 0)),
    )
    return pl.pallas_call(
        _gmm_body,
        grid_spec=grid_spec,
        out_shape=jax.ShapeDtypeStruct((NROW, D_MODEL), jnp.float32),
    )(grp, valid, xg, W1, b1r, W2, b2r)


def _combine_body(og_hbm, rout_hbm, base_hbm, out_hbm,
                  routl, basel, posb1, posb2, buf1, buf2, sem):
    wid = lax.axis_index("s") * 2 + lax.axis_index("c")
    tpw = T // 32
    tok0 = wid * tpw
    pltpu.sync_copy(base_hbm, basel)
    pltpu.sync_copy(rout_hbm.at[pl.ds(tok0, tpw)], routl)

    def jbody(j, _):
        t0 = j * 16
        rows = jnp.arange(16, dtype=jnp.int32) + t0
        ev1 = _rowcol(routl, rows, 0).astype(jnp.int32)
        ev2 = _rowcol(routl, rows, 1).astype(jnp.int32)
        rv1 = _rowcol(routl, rows, 2).astype(jnp.int32)
        rv2 = _rowcol(routl, rows, 3).astype(jnp.int32)
        posb1[...] = plsc.load_gather(basel, [ev1]) + rv1
        posb2[...] = plsc.load_gather(basel, [ev2]) + rv2
        cp1 = pltpu.async_copy(og_hbm.at[posb1], buf1, sem)
        cp2 = pltpu.async_copy(og_hbm.at[posb2], buf2, sem)
        cp1.wait()
        cp2.wait()
        for r in range(16):
            rr = jnp.full((16,), t0 + r, jnp.int32)
            ws1 = plsc.load_gather(routl, [rr, jnp.full((16,), 4, jnp.int32)])
            ws2 = plsc.load_gather(routl, [rr, jnp.full((16,), 5, jnp.int32)])

            def cbody(c, _):
                for u in range(8):
                    sl = pl.ds(c * 128 + u * 16, 16)
                    buf1[r, sl] = buf1[r, sl] * ws1 + buf2[r, sl] * ws2
                return 0

            lax.fori_loop(0, D_MODEL // 128, cbody, 0)
        pltpu.sync_copy(buf1, out_hbm.at[pl.ds(tok0 + t0, 16)])
        return 0

    lax.fori_loop(0, tpw // 16, jbody, 0)


def _combine(og, rout, base16):
    tpw = T // 32
    fn = functools.partial(
        pl.kernel, mesh=_sc_mesh(),
        out_type=jax.ShapeDtypeStruct((T, D_MODEL), jnp.float32),
        scratch_types=[
            pltpu.VMEM((tpw, 128), jnp.float32),
            pltpu.VMEM((16,), jnp.int32),
            pltpu.VMEM((16,), jnp.int32),
            pltpu.VMEM((16,), jnp.int32),
            pltpu.VMEM((16, D_MODEL), jnp.float32),
            pltpu.VMEM((16, D_MODEL), jnp.float32),
            pltpu.SemaphoreType.DMA,
        ],
        compiler_params=pltpu.CompilerParams(needs_layout_passes=False),
    )(_combine_body)
    return fn(og, rout, base16)


def kernel(inputs, Wg, bg, W1, b1, W2, b2):
    i32 = jnp.int32
    gl = inputs @ Wg + bg
    glp = jnp.concatenate(
        [gl, jnp.full((T, 128 - E), NEG, jnp.float32)], axis=1)

    rout, totf = _route(glp)

    sizes = totf[NBLK - 1, 0, :E].astype(i32)
    tpe = (sizes + BT - 1) // BT
    tstart = jnp.concatenate([jnp.zeros((1,), i32), jnp.cumsum(tpe)])[:E]
    base16 = jnp.pad(tstart * BT, (0, 16 - E)).astype(i32)
    grp = (jnp.arange(NTILE, dtype=i32)[:, None]
           >= tstart[None, :]).sum(axis=1).astype(i32) - 1
    valid = (jnp.arange(NTILE, dtype=i32) < jnp.sum(tpe)).astype(i32)

    xg = _dispatch(inputs, rout, base16)
    b1r = b1.reshape(E, 1, D_FF)
    b2r = b2.reshape(E, 1, D_MODEL)
    og = _gmm(grp, valid, xg, W1, b1r, W2, b2r)
    return _combine(og, rout, base16)


# speedup vs baseline: 1.1396x; 1.0067x over previous
import functools

import jax
import jax.numpy as jnp
from jax import lax
from jax.experimental import pallas as pl
from jax.experimental.pallas import tpu as pltpu
from jax.experimental.pallas import tpu_sc as plsc

E = 8
TOP_K = 2
D_MODEL = 2048
D_FF = 4096
T = 8192

BT = 512
NBLK = T // BT
NSLOT = T * TOP_K
NTILE = NSLOT // BT + E - 1
NROW = NTILE * BT
FFB = 1024
NFF = D_FF // FFB
NEG = -1e30


def _route_body(gl_ref, rout_ref, tot_ref, carry):
    b = pl.program_id(0)

    @pl.when(b == 0)
    def _():
        carry[...] = jnp.zeros_like(carry)

    logits = gl_ref[...]
    li = lax.broadcasted_iota(jnp.int32, (BT, 128), 1)

    m1 = jnp.max(logits, axis=1, keepdims=True)
    a1 = jnp.min(jnp.where(logits == m1, li, 128), axis=1, keepdims=True)
    sel1 = li == a1
    logits2 = jnp.where(sel1, NEG, logits)
    m2 = jnp.max(logits2, axis=1, keepdims=True)
    a2 = jnp.min(jnp.where(logits2 == m2, li, 128), axis=1, keepdims=True)
    sel2 = li == a2

    e = jnp.exp(m2 - m1)
    w1v = 1.0 / (1.0 + e)
    w2v = 1.0 - w1v

    oh1 = sel1.astype(jnp.float32)
    oh2 = sel2.astype(jnp.float32)
    ri = lax.broadcasted_iota(jnp.int32, (BT, BT), 0)
    ci = lax.broadcasted_iota(jnp.int32, (BT, BT), 1)
    tril = (ci < ri).astype(jnp.float32)
    ex1 = jnp.dot(tril, oh1, preferred_element_type=jnp.float32)
    ex2 = jnp.dot(tril, oh2, preferred_element_type=jnp.float32)
    cnt1 = jnp.sum(oh1, axis=0, keepdims=True)
    cnt2 = jnp.sum(oh2, axis=0, keepdims=True)
    c0 = carry[...]
    rank1 = jnp.sum(oh1 * (c0 + ex1), axis=1, keepdims=True)
    rank2 = jnp.sum(oh2 * (c0 + cnt1 + ex2), axis=1, keepdims=True)
    cnew = c0 + cnt1 + cnt2
    carry[...] = cnew
    tot_ref[...] = cnew.reshape(1, 1, 128)

    a1f = a1.astype(jnp.float32)
    a2f = a2.astype(jnp.float32)
    packed = (jnp.where(li == 0, a1f, 0.0) + jnp.where(li == 1, a2f, 0.0)
              + jnp.where(li == 2, rank1, 0.0) + jnp.where(li == 3, rank2, 0.0)
              + jnp.where(li == 4, w1v, 0.0) + jnp.where(li == 5, w2v, 0.0))
    rout_ref[...] = packed


def _route(glp):
    return pl.pallas_call(
        _route_body,
        grid=(NBLK,),
        in_specs=[
            pl.BlockSpec((BT, 128), lambda b: (b, 0)),
        ],
        out_specs=[
            pl.BlockSpec((BT, 128), lambda b: (b, 0)),
            pl.BlockSpec((1, 1, 128), lambda b: (b, 0, 0)),
        ],
        out_shape=[
            jax.ShapeDtypeStruct((T, 128), jnp.float32),
            jax.ShapeDtypeStruct((NBLK, 1, 128), jnp.float32),
        ],
        scratch_shapes=[pltpu.VMEM((1, 128), jnp.float32)],
    )(glp)


def _sc_mesh():
    return plsc.VectorSubcoreMesh(core_axis_name="c", subcore_axis_name="s")


def _rowcol(routl, rows, col):
    return plsc.load_gather(routl, [rows, jnp.full((16,), col, jnp.int32)])


def _dispatch_body(x_hbm, rout_hbm, base_hbm, xg_hbm, routl, basel,
                   xbuf0, xbuf1, pa1, pa2, pb1, pb2, rsem, wsem):
    wid = lax.axis_index("s") * 2 + lax.axis_index("c")
    tpw = T // 32
    tok0 = wid * tpw
    nchunk = tpw // 16
    pltpu.sync_copy(base_hbm, basel)
    pltpu.sync_copy(rout_hbm.at[pl.ds(tok0, tpw)], routl)
    xbufs = [xbuf0, xbuf1]
    rd = [None] * nchunk
    wr = [None] * nchunk
    rd[0] = pltpu.async_copy(x_hbm.at[pl.ds(tok0, 16)], xbufs[0], rsem)
    for j in range(nchunk):
        b = j & 1
        t0 = j * 16
        rows = jnp.arange(16, dtype=jnp.int32) + t0
        ev1 = _rowcol(routl, rows, 0).astype(jnp.int32)
        ev2 = _rowcol(routl, rows, 1).astype(jnp.int32)
        rv1 = _rowcol(routl, rows, 2).astype(jnp.int32)
        rv2 = _rowcol(routl, rows, 3).astype(jnp.int32)
        p1 = pa1 if b == 0 else pb1
        p2 = pa2 if b == 0 else pb2
        rd[j].wait()
        if j >= 1:
            for cp in wr[j - 1]:
                cp.wait()
        p1[...] = plsc.load_gather(basel, [ev1]) + rv1
        p2[...] = plsc.load_gather(basel, [ev2]) + rv2
        if j + 1 < nchunk:
            rd[j + 1] = pltpu.async_copy(
                x_hbm.at[pl.ds(tok0 + t0 + 16, 16)], xbufs[1 - b], rsem)
        wr[j] = (pltpu.async_copy(xbufs[b], xg_hbm.at[p1], wsem),
                 pltpu.async_copy(xbufs[b], xg_hbm.at[p2], wsem))
    for cp in wr[nchunk - 1]:
        cp.wait()


def _dispatch(x, rout, base16):
    tpw = T // 32
    fn = functools.partial(
        pl.kernel, mesh=_sc_mesh(),
        out_type=jax.ShapeDtypeStruct((NROW, D_MODEL), jnp.float32),
        scratch_types=[
            pltpu.VMEM((tpw, 128), jnp.float32),
            pltpu.VMEM((16,), jnp.int32),
            pltpu.VMEM((16, D_MODEL), jnp.float32),
            pltpu.VMEM((16, D_MODEL), jnp.float32),
            pltpu.VMEM((16,), jnp.int32),
            pltpu.VMEM((16,), jnp.int32),
            pltpu.VMEM((16,), jnp.int32),
            pltpu.VMEM((16,), jnp.int32),
            pltpu.SemaphoreType.DMA,
            pltpu.SemaphoreType.DMA,
        ],
        compiler_params=pltpu.CompilerParams(needs_layout_passes=False),
    )(_dispatch_body)
    return fn(x, rout, base16)


def _gmm_body(grp_ref, valid_ref, xg_ref, w1_ref, b1_ref, w2_ref, b2_ref,
              og_ref):
    p = pl.program_id(0)
    ff = pl.program_id(1)

    @pl.when(ff == 0)
    def _():
        og_ref[...] = jnp.broadcast_to(b2_ref[0], (BT, D_MODEL))

    @pl.when(valid_ref[p] == 1)
    def _():
        x = xg_ref[...].astype(jnp.bfloat16)
        h = jnp.dot(x, w1_ref[0].astype(jnp.bfloat16),
                    preferred_element_type=jnp.float32)
        h = h + b1_ref[0]
        h = h * jax.nn.sigmoid(h)
        og_ref[...] += jnp.dot(h.astype(jnp.bfloat16),
                               w2_ref[0].astype(jnp.bfloat16),
                               preferred_element_type=jnp.float32)


def _gmm(grp, valid, xg, W1, b1r, W2, b2r):
    grid_spec = pltpu.PrefetchScalarGridSpec(
        num_scalar_prefetch=2,
        grid=(NTILE, NFF),
        in_specs=[
            pl.BlockSpec((BT, D_MODEL), lambda p, ff, g, v: (p, 0)),
            pl.BlockSpec((1, D_MODEL, FFB), lambda p, ff, g, v: (g[p], 0, ff)),
            pl.BlockSpec((1, 1, FFB), lambda p, ff, g, v: (g[p], 0, ff)),
            pl.BlockSpec((1, FFB, D_MODEL), lambda p, ff, g, v: (g[p], ff, 0)),
            pl.BlockSpec((1, 1, D_MODEL), lambda p, ff, g, v: (g[p], 0, 0)),
        ],
        out_specs=pl.BlockSpec((BT, D_MODEL), lambda p, ff, g, v: (p, 0)),
    )
    return pl.pallas_call(
        _gmm_body,
        grid_spec=grid_spec,
        out_shape=jax.ShapeDtypeStruct((NROW, D_MODEL), jnp.float32),
    )(grp, valid, xg, W1, b1r, W2, b2r)


def _combine_body(og_hbm, rout_hbm, base_hbm, out_hbm,
                  routl, basel, posb1, posb2, buf1, buf2, sem):
    wid = lax.axis_index("s") * 2 + lax.axis_index("c")
    tpw = T // 32
    tok0 = wid * tpw
    pltpu.sync_copy(base_hbm, basel)
    pltpu.sync_copy(rout_hbm.at[pl.ds(tok0, tpw)], routl)

    def jbody(j, _):
        t0 = j * 16
        rows = jnp.arange(16, dtype=jnp.int32) + t0
        ev1 = _rowcol(routl, rows, 0).astype(jnp.int32)
        ev2 = _rowcol(routl, rows, 1).astype(jnp.int32)
        rv1 = _rowcol(routl, rows, 2).astype(jnp.int32)
        rv2 = _rowcol(routl, rows, 3).astype(jnp.int32)
        posb1[...] = plsc.load_gather(basel, [ev1]) + rv1
        posb2[...] = plsc.load_gather(basel, [ev2]) + rv2
        cp1 = pltpu.async_copy(og_hbm.at[posb1], buf1, sem)
        cp2 = pltpu.async_copy(og_hbm.at[posb2], buf2, sem)
        cp1.wait()
        cp2.wait()
        for r in range(16):
            rr = jnp.full((16,), t0 + r, jnp.int32)
            ws1 = plsc.load_gather(routl, [rr, jnp.full((16,), 4, jnp.int32)])
            ws2 = plsc.load_gather(routl, [rr, jnp.full((16,), 5, jnp.int32)])

            def cbody(c, _):
                for u in range(8):
                    sl = pl.ds(c * 128 + u * 16, 16)
                    buf1[r, sl] = buf1[r, sl] * ws1 + buf2[r, sl] * ws2
                return 0

            lax.fori_loop(0, D_MODEL // 128, cbody, 0)
        pltpu.sync_copy(buf1, out_hbm.at[pl.ds(tok0 + t0, 16)])
        return 0

    lax.fori_loop(0, tpw // 16, jbody, 0)


def _combine(og, rout, base16):
    tpw = T // 32
    fn = functools.partial(
        pl.kernel, mesh=_sc_mesh(),
        out_type=jax.ShapeDtypeStruct((T, D_MODEL), jnp.float32),
        scratch_types=[
            pltpu.VMEM((tpw, 128), jnp.float32),
            pltpu.VMEM((16,), jnp.int32),
            pltpu.VMEM((16,), jnp.int32),
            pltpu.VMEM((16,), jnp.int32),
            pltpu.VMEM((16, D_MODEL), jnp.float32),
            pltpu.VMEM((16, D_MODEL), jnp.float32),
            pltpu.SemaphoreType.DMA,
        ],
        compiler_params=pltpu.CompilerParams(needs_layout_passes=False),
    )(_combine_body)
    return fn(og, rout, base16)


def kernel(inputs, Wg, bg, W1, b1, W2, b2):
    i32 = jnp.int32
    gl = inputs @ Wg + bg
    glp = jnp.concatenate(
        [gl, jnp.full((T, 128 - E), NEG, jnp.float32)], axis=1)

    rout, totf = _route(glp)

    sizes = totf[NBLK - 1, 0, :E].astype(i32)
    tpe = (sizes + BT - 1) // BT
    tstart = jnp.concatenate([jnp.zeros((1,), i32), jnp.cumsum(tpe)])[:E]
    base16 = jnp.pad(tstart * BT, (0, 16 - E)).astype(i32)
    grp = (jnp.arange(NTILE, dtype=i32)[:, None]
           >= tstart[None, :]).sum(axis=1).astype(i32) - 1
    valid = (jnp.arange(NTILE, dtype=i32) < jnp.sum(tpe)).astype(i32)

    xg = _dispatch(inputs, rout, base16)
    b1r = b1.reshape(E, 1, D_FF)
    b2r = b2.reshape(E, 1, D_MODEL)
    og = _gmm(grp, valid, xg, W1, b1r, W2, b2r)
    return _combine(og, rout, base16)
